# Initial kernel scaffold; baseline (speedup 1.0000x reference)
import jax
import jax.numpy as jnp
from jax.experimental import pallas as pl


def kernel(x, edge_index, batch, W_node, b_node, Wg0, bg0, Wg1, bg1, Wg2, bg2, Wfc1, bfc1, Wfc2, bfc2):
    raise NotImplementedError("write your pallas kernel here")


# speedup vs baseline: 7.3219x; 7.3219x over previous
import dataclasses
import functools

import jax
import jax.numpy as jnp
from jax import lax
from jax.experimental import pallas as pl
from jax.experimental.pallas import tpu as pltpu
from jax.experimental.pallas import tpu_sc as plsc

NN = 10000
NP = 10240
EE = 320000
EP = 327680
DIN = 128
HH = 64
HP = 128
GG = 16
NC = 2
NS = 16
ROWS_PER_TILE = EP // 128 // (NC * NS)
CHUNK_ROWS = 2
N_CHUNKS = ROWS_PER_TILE // CHUNK_ROWS
ROWS_PER_SUB = NP // NS
BR = 512
NB = NP // BR

_mesh = plsc.VectorSubcoreMesh(core_axis_name="c", subcore_axis_name="s",
                               num_cores=NC, num_subcores=NS)

_sc_params = pltpu.CompilerParams()
if "needs_layout_passes" in pltpu.CompilerParams.__dataclass_fields__:
    _sc_params = dataclasses.replace(_sc_params, needs_layout_passes=False)


@functools.partial(
    pl.kernel,
    out_type=jax.ShapeDtypeStruct((NC * NS, NP), jnp.float32),
    mesh=_mesh,
    scratch_types=[
        pltpu.VMEM((NP,), jnp.float32),
        pltpu.VMEM((8, 128), jnp.int32),
    ],
    compiler_params=_sc_params,
)
def _sc_deg(dst_hbm, out_hbm, hist, idxs):
    cid = lax.axis_index("c")
    sid = lax.axis_index("s")
    wid = sid * NC + cid

    @pl.loop(0, NP, step=16)
    def _zero(i):
        hist[pl.ds(i, 16)] = jnp.zeros((16,), jnp.float32)

    ones = jnp.ones((16,), jnp.float32)
    base = wid * ROWS_PER_TILE

    @pl.loop(0, ROWS_PER_TILE, step=8)
    def _chunk(r):
        pltpu.sync_copy(dst_hbm.at[pl.ds(base + r, 8)], idxs)
        for j in range(8):
            for k in range(8):
                idxv = idxs[j, pl.ds(k * 16, 16)]
                plsc.addupdate_scatter(hist, [idxv], ones)

    pltpu.sync_copy(hist, out_hbm.at[wid])


@functools.partial(
    pl.kernel,
    out_type=jax.ShapeDtypeStruct((NC, NP, HP), jnp.float32),
    mesh=_mesh,
    scratch_types=[
        pltpu.VMEM_SHARED((NP, HP), jnp.float32),
        pltpu.VMEM((CHUNK_ROWS, 128), jnp.int32),
        pltpu.VMEM((CHUNK_ROWS, 128), jnp.int32),
        pltpu.VMEM((CHUNK_ROWS * 128, HP), jnp.float32),
        pltpu.SemaphoreType.DMA,
    ],
    compiler_params=_sc_params,
)
def _sc_agg(src_hbm, dst_hbm, hwp_hbm, zeros_hbm, out_hbm,
            acc, idx_src, idx_dst, rows, sem):
    cid = lax.axis_index("c")
    sid = lax.axis_index("s")
    sl = pl.ds(sid * ROWS_PER_SUB, ROWS_PER_SUB)

    pltpu.sync_copy(zeros_hbm.at[sl], acc.at[sl])
    plsc.subcore_barrier()

    base = (cid * NS + sid) * ROWS_PER_TILE

    @pl.loop(0, N_CHUNKS)
    def _chunk(ch):
        r = base + ch * CHUNK_ROWS
        pltpu.sync_copy(src_hbm.at[pl.ds(r, CHUNK_ROWS)], idx_src)
        pltpu.sync_copy(dst_hbm.at[pl.ds(r, CHUNK_ROWS)], idx_dst)
        copies = [
            pltpu.async_copy(hwp_hbm.at[idx_src.at[j]],
                             rows.at[pl.ds(j * 128, 128)], sem)
            for j in range(CHUNK_ROWS)
        ]
        for c in copies:
            c.wait()
        for j in range(CHUNK_ROWS):
            pltpu.sync_copy(rows.at[pl.ds(j * 128, 128)],
                            acc.at[idx_dst.at[j]], add=True)

    plsc.subcore_barrier()
    pltpu.sync_copy(acc.at[sl], out_hbm.at[cid, sl])


def _pad_cols(h):
    return jnp.concatenate([h, jnp.zeros((h.shape[0], HP - HH), jnp.float32)],
                           axis=1)


def _stage_a_body(x_ref, degp_ref, wn_ref, wg_ref, bn_ref, hwp_ref, dinv_ref):
    deg = jnp.sum(degp_ref[...], axis=0) + 1.0
    dinv = lax.rsqrt(deg)
    d64 = jnp.broadcast_to(dinv[:, None], (BR, HH))
    h0 = jnp.maximum(
        jnp.dot(x_ref[...], wn_ref[...], preferred_element_type=jnp.float32)
        + bn_ref[0][None, :], 0.0)
    hwp_ref[...] = _pad_cols(
        jnp.dot(h0, wg_ref[...], preferred_element_type=jnp.float32) * d64)
    dinv_ref[...] = d64


def _stage_a(xp, degp, wn, wg0, bn8):
    return pl.pallas_call(
        _stage_a_body,
        grid=(NB,),
        in_specs=[
            pl.BlockSpec((BR, DIN), lambda i: (i, 0)),
            pl.BlockSpec((NC * NS, BR), lambda i: (0, i)),
            pl.BlockSpec((DIN, HH), lambda i: (0, 0)),
            pl.BlockSpec((HH, HH), lambda i: (0, 0)),
            pl.BlockSpec((8, HH), lambda i: (0, 0)),
        ],
        out_specs=[
            pl.BlockSpec((BR, HP), lambda i: (i, 0)),
            pl.BlockSpec((BR, HH), lambda i: (i, 0)),
        ],
        out_shape=[
            jax.ShapeDtypeStruct((NP, HP), jnp.float32),
            jax.ShapeDtypeStruct((NP, HH), jnp.float32),
        ],
    )(xp, degp, wn, wg0, bn8)


def _stage_b_body(p_ref, hwp_ref, dinv_ref, w_ref, b_ref, out_ref):
    d = dinv_ref[...]
    agg = p_ref[0, :, :HH] + p_ref[1, :, :HH] + hwp_ref[:, :HH]
    h = jnp.maximum(d * agg + b_ref[0][None, :], 0.0)
    out_ref[...] = _pad_cols(
        jnp.dot(h, w_ref[...], preferred_element_type=jnp.float32) * d)


def _stage_b(p, hwp, dinv64, w_next, b8):
    return pl.pallas_call(
        _stage_b_body,
        grid=(NB,),
        in_specs=[
            pl.BlockSpec((NC, BR, HP), lambda i: (0, i, 0)),
            pl.BlockSpec((BR, HP), lambda i: (i, 0)),
            pl.BlockSpec((BR, HH), lambda i: (i, 0)),
            pl.BlockSpec((HH, HH), lambda i: (0, 0)),
            pl.BlockSpec((8, HH), lambda i: (0, 0)),
        ],
        out_specs=pl.BlockSpec((BR, HP), lambda i: (i, 0)),
        out_shape=jax.ShapeDtypeStruct((NP, HP), jnp.float32),
    )(p, hwp, dinv64, w_next, b8)


def _stage_c_body(p_ref, hwp_ref, dinv_ref, bg_ref, batch_ref,
                  w1_ref, b1_ref, w2_ref, b2_ref, out_ref, pool_acc, cnt_acc):
    i = pl.program_id(0)

    @pl.when(i == 0)
    def _init():
        pool_acc[...] = jnp.zeros((GG, HH), jnp.float32)
        cnt_acc[...] = jnp.zeros((GG, HH), jnp.float32)

    agg = p_ref[0, :, :HH] + p_ref[1, :, :HH] + hwp_ref[:, :HH]
    h3 = dinv_ref[...] * agg + bg_ref[0][None, :]
    b_ids = batch_ref[0, 0, :]
    iota_g = lax.broadcasted_iota(jnp.int32, (BR, GG), 1)
    onehot = (b_ids[:, None] == iota_g).astype(jnp.float32)
    pool_acc[...] += lax.dot_general(onehot, h3, (((0,), (0,)), ((), ())),
                                     preferred_element_type=jnp.float32)
    cnt_acc[...] += jnp.broadcast_to(jnp.sum(onehot, axis=0)[:, None], (GG, HH))

    @pl.when(i == NB - 1)
    def _fin():
        pooled = pool_acc[...] / jnp.maximum(cnt_acc[...], 1.0)
        hh = jnp.maximum(
            jnp.dot(pooled, w1_ref[...], preferred_element_type=jnp.float32)
            + b1_ref[0][None, :], 0.0)
        out_ref[...] = jnp.dot(hh, w2_ref[...],
                               preferred_element_type=jnp.float32) \
            + b2_ref[0][None, :]


def _stage_c(p, hwp, dinv64, bg8, batch3d, w1, b18, w2p, b2p8):
    return pl.pallas_call(
        _stage_c_body,
        grid=(NB,),
        in_specs=[
            pl.BlockSpec((NC, BR, HP), lambda i: (0, i, 0)),
            pl.BlockSpec((BR, HP), lambda i: (i, 0)),
            pl.BlockSpec((BR, HH), lambda i: (i, 0)),
            pl.BlockSpec((8, HH), lambda i: (0, 0)),
            pl.BlockSpec((1, 1, BR), lambda i: (i, 0, 0)),
            pl.BlockSpec((HH, HH), lambda i: (0, 0)),
            pl.BlockSpec((8, HH), lambda i: (0, 0)),
            pl.BlockSpec((HH, 128), lambda i: (0, 0)),
            pl.BlockSpec((8, 128), lambda i: (0, 0)),
        ],
        out_specs=pl.BlockSpec((GG, 128), lambda i: (0, 0)),
        out_shape=jax.ShapeDtypeStruct((GG, 128), jnp.float32),
        scratch_shapes=[
            pltpu.VMEM((GG, HH), jnp.float32),
            pltpu.VMEM((GG, HH), jnp.float32),
        ],
    )(p, hwp, dinv64, bg8, batch3d, w1, b18, w2p, b2p8)


def kernel(x, edge_index, batch, W_node, b_node, Wg0, bg0, Wg1, bg1, Wg2, bg2,
           Wfc1, bfc1, Wfc2, bfc2):
    f32 = jnp.float32
    pad_idx = jnp.full((EP - EE,), NN, jnp.int32)
    src2d = jnp.concatenate([edge_index[0], pad_idx]).reshape(EP // 128, 128)
    dst2d = jnp.concatenate([edge_index[1], pad_idx]).reshape(EP // 128, 128)
    xp = jnp.concatenate([x, jnp.zeros((NP - NN, DIN), f32)])
    batch3d = jnp.concatenate(
        [batch, jnp.full((NP - NN,), GG, jnp.int32)]).reshape(NB, 1, BR)
    zeros_nh = jnp.zeros((NP, HP), f32)

    bn8 = jnp.broadcast_to(b_node[None, :], (8, HH))
    bg0_8 = jnp.broadcast_to(bg0[None, :], (8, HH))
    bg1_8 = jnp.broadcast_to(bg1[None, :], (8, HH))
    bg2_8 = jnp.broadcast_to(bg2[None, :], (8, HH))
    bfc1_8 = jnp.broadcast_to(bfc1[None, :], (8, HH))
    w2p = jnp.zeros((HH, 128), f32).at[:, :1].set(Wfc2)
    b2p8 = jnp.broadcast_to(
        jnp.concatenate([bfc2, jnp.zeros((127,), f32)])[None, :], (8, 128))

    degp = _sc_deg(dst2d)
    hwp0, dinv64 = _stage_a(xp, degp, W_node, Wg0, bn8)
    p = _sc_agg(src2d, dst2d, hwp0, zeros_nh)
    hwp1 = _stage_b(p, hwp0, dinv64, Wg1, bg0_8)
    p = _sc_agg(src2d, dst2d, hwp1, zeros_nh)
    hwp2 = _stage_b(p, hwp1, dinv64, Wg2, bg1_8)
    p = _sc_agg(src2d, dst2d, hwp2, zeros_nh)
    outp = _stage_c(p, hwp2, dinv64, bg2_8, batch3d, Wfc1, bfc1_8, w2p, b2p8)
    return outp[:GG, :1]


# speedup vs baseline: 8.3759x; 1.1440x over previous
import dataclasses
import functools

import jax
import jax.numpy as jnp
from jax import lax
from jax.experimental import pallas as pl
from jax.experimental.pallas import tpu as pltpu
from jax.experimental.pallas import tpu_sc as plsc

NN = 10000
NP = 10240
EE = 320000
EP = 327680
DIN = 128
HH = 64
HP = 128
GG = 16
NC = 2
NS = 16
ROWS_PER_TILE = EP // 128 // (NC * NS)
CPS = 16
SUP = ROWS_PER_TILE // CPS
ROWS_PER_SUB = NP // NS
BR = 512
NB = NP // BR

_mesh = plsc.VectorSubcoreMesh(core_axis_name="c", subcore_axis_name="s",
                               num_cores=NC, num_subcores=NS)

_sc_params = pltpu.CompilerParams()
if "needs_layout_passes" in pltpu.CompilerParams.__dataclass_fields__:
    _sc_params = dataclasses.replace(_sc_params, needs_layout_passes=False)


@functools.partial(
    pl.kernel,
    out_type=jax.ShapeDtypeStruct((NC * NS, NP), jnp.float32),
    mesh=_mesh,
    scratch_types=[
        pltpu.VMEM((NP,), jnp.float32),
        pltpu.VMEM((8, 128), jnp.int32),
    ],
    compiler_params=_sc_params,
)
def _sc_deg(dst_hbm, out_hbm, hist, idxs):
    cid = lax.axis_index("c")
    sid = lax.axis_index("s")
    wid = sid * NC + cid

    @pl.loop(0, NP, step=16)
    def _zero(i):
        hist[pl.ds(i, 16)] = jnp.zeros((16,), jnp.float32)

    ones = jnp.ones((16,), jnp.float32)
    base = wid * ROWS_PER_TILE

    @pl.loop(0, ROWS_PER_TILE, step=8)
    def _chunk(r):
        pltpu.sync_copy(dst_hbm.at[pl.ds(base + r, 8)], idxs)
        for j in range(8):
            for k in range(8):
                idxv = idxs[j, pl.ds(k * 16, 16)]
                plsc.addupdate_scatter(hist, [idxv], ones)

    pltpu.sync_copy(hist, out_hbm.at[wid])


@functools.partial(
    pl.kernel,
    out_type=jax.ShapeDtypeStruct((NC, NP, HP), jnp.float32),
    mesh=_mesh,
    scratch_types=[
        pltpu.VMEM_SHARED((NP, HP), jnp.float32),
        pltpu.VMEM((CPS, 128), jnp.int32),
        pltpu.VMEM((CPS, 128), jnp.int32),
        pltpu.VMEM((128, HP), jnp.float32),
        pltpu.VMEM((128, HP), jnp.float32),
        pltpu.SemaphoreType.DMA,
        pltpu.SemaphoreType.DMA,
    ],
    compiler_params=_sc_params,
)
def _sc_agg(src_hbm, dst_hbm, hwp_hbm, zeros_hbm, out_hbm,
            acc, idx_src, idx_dst, rows_a, rows_b, sem_a, sem_b):
    cid = lax.axis_index("c")
    sid = lax.axis_index("s")
    sl = pl.ds(sid * ROWS_PER_SUB, ROWS_PER_SUB)

    pltpu.sync_copy(zeros_hbm.at[sl], acc.at[sl])
    plsc.subcore_barrier()

    base = (cid * NS + sid) * ROWS_PER_TILE
    rows = (rows_a, rows_b)
    sems = (sem_a, sem_b)

    @pl.loop(0, SUP)
    def _sup(s):
        r = base + s * CPS
        pltpu.sync_copy(src_hbm.at[pl.ds(r, CPS)], idx_src)
        pltpu.sync_copy(dst_hbm.at[pl.ds(r, CPS)], idx_dst)
        descs = [None, None]
        for j in range(CPS):
            b = j % 2
            descs[b] = pltpu.async_copy(hwp_hbm.at[idx_src.at[j]],
                                        rows[b], sems[b])
            if j >= 1:
                descs[1 - b].wait()
                pltpu.sync_copy(rows[1 - b], acc.at[idx_dst.at[j - 1]],
                                add=True)
        descs[(CPS - 1) % 2].wait()
        pltpu.sync_copy(rows[(CPS - 1) % 2], acc.at[idx_dst.at[CPS - 1]],
                        add=True)

    plsc.subcore_barrier()
    pltpu.sync_copy(acc.at[sl], out_hbm.at[cid, sl])


def _pad_cols(h):
    return jnp.concatenate([h, jnp.zeros((h.shape[0], HP - HH), jnp.float32)],
                           axis=1)


def _stage_a_body(x_ref, degp_ref, wn_ref, wg_ref, bn_ref, hwp_ref, dinv_ref):
    deg = jnp.sum(degp_ref[...], axis=0) + 1.0
    dinv = lax.rsqrt(deg)
    d64 = jnp.broadcast_to(dinv[:, None], (BR, HH))
    h0 = jnp.maximum(
        jnp.dot(x_ref[...], wn_ref[...], preferred_element_type=jnp.float32)
        + bn_ref[0][None, :], 0.0)
    hwp_ref[...] = _pad_cols(
        jnp.dot(h0, wg_ref[...], preferred_element_type=jnp.float32) * d64)
    dinv_ref[...] = d64


def _stage_a(xp, degp, wn, wg0, bn8):
    return pl.pallas_call(
        _stage_a_body,
        grid=(NB,),
        in_specs=[
            pl.BlockSpec((BR, DIN), lambda i: (i, 0)),
            pl.BlockSpec((NC * NS, BR), lambda i: (0, i)),
            pl.BlockSpec((DIN, HH), lambda i: (0, 0)),
            pl.BlockSpec((HH, HH), lambda i: (0, 0)),
            pl.BlockSpec((8, HH), lambda i: (0, 0)),
        ],
        out_specs=[
            pl.BlockSpec((BR, HP), lambda i: (i, 0)),
            pl.BlockSpec((BR, HH), lambda i: (i, 0)),
        ],
        out_shape=[
            jax.ShapeDtypeStruct((NP, HP), jnp.float32),
            jax.ShapeDtypeStruct((NP, HH), jnp.float32),
        ],
    )(xp, degp, wn, wg0, bn8)


def _stage_b_body(p_ref, hwp_ref, dinv_ref, w_ref, b_ref, out_ref):
    d = dinv_ref[...]
    agg = p_ref[0, :, :HH] + p_ref[1, :, :HH] + hwp_ref[:, :HH]
    h = jnp.maximum(d * agg + b_ref[0][None, :], 0.0)
    out_ref[...] = _pad_cols(
        jnp.dot(h, w_ref[...], preferred_element_type=jnp.float32) * d)


def _stage_b(p, hwp, dinv64, w_next, b8):
    return pl.pallas_call(
        _stage_b_body,
        grid=(NB,),
        in_specs=[
            pl.BlockSpec((NC, BR, HP), lambda i: (0, i, 0)),
            pl.BlockSpec((BR, HP), lambda i: (i, 0)),
            pl.BlockSpec((BR, HH), lambda i: (i, 0)),
            pl.BlockSpec((HH, HH), lambda i: (0, 0)),
            pl.BlockSpec((8, HH), lambda i: (0, 0)),
        ],
        out_specs=pl.BlockSpec((BR, HP), lambda i: (i, 0)),
        out_shape=jax.ShapeDtypeStruct((NP, HP), jnp.float32),
    )(p, hwp, dinv64, w_next, b8)


def _stage_c_body(p_ref, hwp_ref, dinv_ref, bg_ref, batch_ref,
                  w1_ref, b1_ref, w2_ref, b2_ref, out_ref, pool_acc, cnt_acc):
    i = pl.program_id(0)

    @pl.when(i == 0)
    def _init():
        pool_acc[...] = jnp.zeros((GG, HH), jnp.float32)
        cnt_acc[...] = jnp.zeros((GG, HH), jnp.float32)

    agg = p_ref[0, :, :HH] + p_ref[1, :, :HH] + hwp_ref[:, :HH]
    h3 = dinv_ref[...] * agg + bg_ref[0][None, :]
    b_ids = batch_ref[0, 0, :]
    iota_g = lax.broadcasted_iota(jnp.int32, (BR, GG), 1)
    onehot = (b_ids[:, None] == iota_g).astype(jnp.float32)
    pool_acc[...] += lax.dot_general(onehot, h3, (((0,), (0,)), ((), ())),
                                     preferred_element_type=jnp.float32)
    cnt_acc[...] += jnp.broadcast_to(jnp.sum(onehot, axis=0)[:, None], (GG, HH))

    @pl.when(i == NB - 1)
    def _fin():
        pooled = pool_acc[...] / jnp.maximum(cnt_acc[...], 1.0)
        hh = jnp.maximum(
            jnp.dot(pooled, w1_ref[...], preferred_element_type=jnp.float32)
            + b1_ref[0][None, :], 0.0)
        out_ref[...] = jnp.dot(hh, w2_ref[...],
                               preferred_element_type=jnp.float32) \
            + b2_ref[0][None, :]


def _stage_c(p, hwp, dinv64, bg8, batch3d, w1, b18, w2p, b2p8):
    return pl.pallas_call(
        _stage_c_body,
        grid=(NB,),
        in_specs=[
            pl.BlockSpec((NC, BR, HP), lambda i: (0, i, 0)),
            pl.BlockSpec((BR, HP), lambda i: (i, 0)),
            pl.BlockSpec((BR, HH), lambda i: (i, 0)),
            pl.BlockSpec((8, HH), lambda i: (0, 0)),
            pl.BlockSpec((1, 1, BR), lambda i: (i, 0, 0)),
            pl.BlockSpec((HH, HH), lambda i: (0, 0)),
            pl.BlockSpec((8, HH), lambda i: (0, 0)),
            pl.BlockSpec((HH, 128), lambda i: (0, 0)),
            pl.BlockSpec((8, 128), lambda i: (0, 0)),
        ],
        out_specs=pl.BlockSpec((GG, 128), lambda i: (0, 0)),
        out_shape=jax.ShapeDtypeStruct((GG, 128), jnp.float32),
        scratch_shapes=[
            pltpu.VMEM((GG, HH), jnp.float32),
            pltpu.VMEM((GG, HH), jnp.float32),
        ],
    )(p, hwp, dinv64, bg8, batch3d, w1, b18, w2p, b2p8)


def kernel(x, edge_index, batch, W_node, b_node, Wg0, bg0, Wg1, bg1, Wg2, bg2,
           Wfc1, bfc1, Wfc2, bfc2):
    f32 = jnp.float32
    pad_idx = jnp.full((EP - EE,), NN, jnp.int32)
    src2d = jnp.concatenate([edge_index[0], pad_idx]).reshape(EP // 128, 128)
    dst2d = jnp.concatenate([edge_index[1], pad_idx]).reshape(EP // 128, 128)
    xp = jnp.concatenate([x, jnp.zeros((NP - NN, DIN), f32)])
    batch3d = jnp.concatenate(
        [batch, jnp.full((NP - NN,), GG, jnp.int32)]).reshape(NB, 1, BR)
    zeros_nh = jnp.zeros((NP, HP), f32)

    bn8 = jnp.broadcast_to(b_node[None, :], (8, HH))
    bg0_8 = jnp.broadcast_to(bg0[None, :], (8, HH))
    bg1_8 = jnp.broadcast_to(bg1[None, :], (8, HH))
    bg2_8 = jnp.broadcast_to(bg2[None, :], (8, HH))
    bfc1_8 = jnp.broadcast_to(bfc1[None, :], (8, HH))
    w2p = jnp.zeros((HH, 128), f32).at[:, :1].set(Wfc2)
    b2p8 = jnp.broadcast_to(
        jnp.concatenate([bfc2, jnp.zeros((127,), f32)])[None, :], (8, 128))

    degp = _sc_deg(dst2d)
    hwp0, dinv64 = _stage_a(xp, degp, W_node, Wg0, bn8)
    p = _sc_agg(src2d, dst2d, hwp0, zeros_nh)
    hwp1 = _stage_b(p, hwp0, dinv64, Wg1, bg0_8)
    p = _sc_agg(src2d, dst2d, hwp1, zeros_nh)
    hwp2 = _stage_b(p, hwp1, dinv64, Wg2, bg1_8)
    p = _sc_agg(src2d, dst2d, hwp2, zeros_nh)
    outp = _stage_c(p, hwp2, dinv64, bg2_8, batch3d, Wfc1, bfc1_8, w2p, b2p8)
    return outp[:GG, :1]


# speedup vs baseline: 8.9286x; 1.0660x over previous
import dataclasses
import functools

import jax
import jax.numpy as jnp
from jax import lax
from jax.experimental import pallas as pl
from jax.experimental.pallas import tpu as pltpu
from jax.experimental.pallas import tpu_sc as plsc

NN = 10000
NP = 10240
EE = 320000
EP = 327680
DIN = 128
HH = 64
HP = 128
GG = 16
NC = 2
NS = 16
ROWS_PER_TILE = EP // 128 // (NC * NS)
R0 = 120
R1 = 40
CPS = 8
SUP0 = R0 // CPS
SUP1 = R1 // CPS
ROWS_PER_SUB = NP // NS
BR = 512
NB = NP // BR

_mesh = plsc.VectorSubcoreMesh(core_axis_name="c", subcore_axis_name="s",
                               num_cores=NC, num_subcores=NS)

_sc_params = pltpu.CompilerParams()
if "needs_layout_passes" in pltpu.CompilerParams.__dataclass_fields__:
    _sc_params = dataclasses.replace(_sc_params, needs_layout_passes=False)


@functools.partial(
    pl.kernel,
    out_type=jax.ShapeDtypeStruct((NC * NS, NP), jnp.float32),
    mesh=_mesh,
    scratch_types=[
        pltpu.VMEM((NP,), jnp.float32),
        pltpu.VMEM((8, 128), jnp.int32),
    ],
    compiler_params=_sc_params,
)
def _sc_deg(dst_hbm, out_hbm, hist, idxs):
    cid = lax.axis_index("c")
    sid = lax.axis_index("s")
    wid = sid * NC + cid

    @pl.loop(0, NP, step=16)
    def _zero(i):
        hist[pl.ds(i, 16)] = jnp.zeros((16,), jnp.float32)

    ones = jnp.ones((16,), jnp.float32)
    base = wid * ROWS_PER_TILE

    @pl.loop(0, ROWS_PER_TILE, step=8)
    def _chunk(r):
        pltpu.sync_copy(dst_hbm.at[pl.ds(base + r, 8)], idxs)
        for j in range(8):
            for k in range(8):
                idxv = idxs[j, pl.ds(k * 16, 16)]
                plsc.addupdate_scatter(hist, [idxv], ones)

    pltpu.sync_copy(hist, out_hbm.at[wid])


@functools.partial(
    pl.kernel,
    out_type=jax.ShapeDtypeStruct((NC, NP, HP), jnp.float32),
    mesh=_mesh,
    scratch_types=[
        pltpu.VMEM_SHARED((NP, HP), jnp.float32),
        pltpu.VMEM((CPS, 128), jnp.int32),
        pltpu.VMEM((CPS, 128), jnp.int32),
        pltpu.VMEM((128, HP), jnp.float32),
        pltpu.VMEM((128, HP), jnp.float32),
        pltpu.SemaphoreType.DMA,
        pltpu.SemaphoreType.DMA,
    ],
    compiler_params=_sc_params,
)
def _sc_agg(src_hbm, dst_hbm, hwp_hbm, zeros_hbm, out_hbm,
            acc, idx_src, idx_dst, rows_a, rows_b, sem_a, sem_b):
    cid = lax.axis_index("c")
    sid = lax.axis_index("s")
    sl = pl.ds(sid * ROWS_PER_SUB, ROWS_PER_SUB)

    pltpu.sync_copy(zeros_hbm.at[sl], acc.at[sl])
    plsc.subcore_barrier()

    rows_tile = jnp.where(cid == 0, R0, R1)
    base = cid * (NS * R0) + sid * rows_tile
    n_sup = jnp.where(cid == 0, SUP0, SUP1)
    rows = (rows_a, rows_b)
    sems = (sem_a, sem_b)

    @pl.loop(0, n_sup)
    def _sup(s):
        r = base + s * CPS
        pltpu.sync_copy(src_hbm.at[pl.ds(r, CPS)], idx_src)
        pltpu.sync_copy(dst_hbm.at[pl.ds(r, CPS)], idx_dst)
        descs = [None, None]
        for j in range(CPS):
            b = j % 2
            descs[b] = pltpu.async_copy(hwp_hbm.at[idx_src.at[j]],
                                        rows[b], sems[b])
            if j >= 1:
                descs[1 - b].wait()
                pltpu.sync_copy(rows[1 - b], acc.at[idx_dst.at[j - 1]],
                                add=True)
        descs[(CPS - 1) % 2].wait()
        pltpu.sync_copy(rows[(CPS - 1) % 2], acc.at[idx_dst.at[CPS - 1]],
                        add=True)

    plsc.subcore_barrier()
    pltpu.sync_copy(acc.at[sl], out_hbm.at[cid, sl])


def _pad_cols(h):
    return jnp.concatenate([h, jnp.zeros((h.shape[0], HP - HH), jnp.float32)],
                           axis=1)


def _stage_a_body(x_ref, degp_ref, wn_ref, wg_ref, bn_ref, hwp_ref, dinv_ref):
    deg = jnp.sum(degp_ref[...], axis=0) + 1.0
    dinv = lax.rsqrt(deg)
    d64 = jnp.broadcast_to(dinv[:, None], (BR, HH))
    h0 = jnp.maximum(
        jnp.dot(x_ref[...], wn_ref[...], preferred_element_type=jnp.float32)
        + bn_ref[0][None, :], 0.0)
    hwp_ref[...] = _pad_cols(
        jnp.dot(h0, wg_ref[...], preferred_element_type=jnp.float32) * d64)
    dinv_ref[...] = d64


def _stage_a(xp, degp, wn, wg0, bn8):
    return pl.pallas_call(
        _stage_a_body,
        grid=(NB,),
        in_specs=[
            pl.BlockSpec((BR, DIN), lambda i: (i, 0)),
            pl.BlockSpec((NC * NS, BR), lambda i: (0, i)),
            pl.BlockSpec((DIN, HH), lambda i: (0, 0)),
            pl.BlockSpec((HH, HH), lambda i: (0, 0)),
            pl.BlockSpec((8, HH), lambda i: (0, 0)),
        ],
        out_specs=[
            pl.BlockSpec((BR, HP), lambda i: (i, 0)),
            pl.BlockSpec((BR, HH), lambda i: (i, 0)),
        ],
        out_shape=[
            jax.ShapeDtypeStruct((NP, HP), jnp.float32),
            jax.ShapeDtypeStruct((NP, HH), jnp.float32),
        ],
    )(xp, degp, wn, wg0, bn8)


def _stage_b_body(p_ref, hwp_ref, dinv_ref, w_ref, b_ref, out_ref):
    d = dinv_ref[...]
    agg = p_ref[0, :, :HH] + p_ref[1, :, :HH] + hwp_ref[:, :HH]
    h = jnp.maximum(d * agg + b_ref[0][None, :], 0.0)
    out_ref[...] = _pad_cols(
        jnp.dot(h, w_ref[...], preferred_element_type=jnp.float32) * d)


def _stage_b(p, hwp, dinv64, w_next, b8):
    return pl.pallas_call(
        _stage_b_body,
        grid=(NB,),
        in_specs=[
            pl.BlockSpec((NC, BR, HP), lambda i: (0, i, 0)),
            pl.BlockSpec((BR, HP), lambda i: (i, 0)),
            pl.BlockSpec((BR, HH), lambda i: (i, 0)),
            pl.BlockSpec((HH, HH), lambda i: (0, 0)),
            pl.BlockSpec((8, HH), lambda i: (0, 0)),
        ],
        out_specs=pl.BlockSpec((BR, HP), lambda i: (i, 0)),
        out_shape=jax.ShapeDtypeStruct((NP, HP), jnp.float32),
    )(p, hwp, dinv64, w_next, b8)


def _stage_c_body(p_ref, hwp_ref, dinv_ref, bg_ref, batch_ref,
                  w1_ref, b1_ref, w2_ref, b2_ref, out_ref, pool_acc, cnt_acc):
    i = pl.program_id(0)

    @pl.when(i == 0)
    def _init():
        pool_acc[...] = jnp.zeros((GG, HH), jnp.float32)
        cnt_acc[...] = jnp.zeros((GG, HH), jnp.float32)

    agg = p_ref[0, :, :HH] + p_ref[1, :, :HH] + hwp_ref[:, :HH]
    h3 = dinv_ref[...] * agg + bg_ref[0][None, :]
    b_ids = batch_ref[0, 0, :]
    iota_g = lax.broadcasted_iota(jnp.int32, (BR, GG), 1)
    onehot = (b_ids[:, None] == iota_g).astype(jnp.float32)
    pool_acc[...] += lax.dot_general(onehot, h3, (((0,), (0,)), ((), ())),
                                     preferred_element_type=jnp.float32)
    cnt_acc[...] += jnp.broadcast_to(jnp.sum(onehot, axis=0)[:, None], (GG, HH))

    @pl.when(i == NB - 1)
    def _fin():
        pooled = pool_acc[...] / jnp.maximum(cnt_acc[...], 1.0)
        hh = jnp.maximum(
            jnp.dot(pooled, w1_ref[...], preferred_element_type=jnp.float32)
            + b1_ref[0][None, :], 0.0)
        out_ref[...] = jnp.dot(hh, w2_ref[...],
                               preferred_element_type=jnp.float32) \
            + b2_ref[0][None, :]


def _stage_c(p, hwp, dinv64, bg8, batch3d, w1, b18, w2p, b2p8):
    return pl.pallas_call(
        _stage_c_body,
        grid=(NB,),
        in_specs=[
            pl.BlockSpec((NC, BR, HP), lambda i: (0, i, 0)),
            pl.BlockSpec((BR, HP), lambda i: (i, 0)),
            pl.BlockSpec((BR, HH), lambda i: (i, 0)),
            pl.BlockSpec((8, HH), lambda i: (0, 0)),
            pl.BlockSpec((1, 1, BR), lambda i: (i, 0, 0)),
            pl.BlockSpec((HH, HH), lambda i: (0, 0)),
            pl.BlockSpec((8, HH), lambda i: (0, 0)),
            pl.BlockSpec((HH, 128), lambda i: (0, 0)),
            pl.BlockSpec((8, 128), lambda i: (0, 0)),
        ],
        out_specs=pl.BlockSpec((GG, 128), lambda i: (0, 0)),
        out_shape=jax.ShapeDtypeStruct((GG, 128), jnp.float32),
        scratch_shapes=[
            pltpu.VMEM((GG, HH), jnp.float32),
            pltpu.VMEM((GG, HH), jnp.float32),
        ],
    )(p, hwp, dinv64, bg8, batch3d, w1, b18, w2p, b2p8)


def kernel(x, edge_index, batch, W_node, b_node, Wg0, bg0, Wg1, bg1, Wg2, bg2,
           Wfc1, bfc1, Wfc2, bfc2):
    f32 = jnp.float32
    pad_idx = jnp.full((EP - EE,), NN, jnp.int32)
    src2d = jnp.concatenate([edge_index[0], pad_idx]).reshape(EP // 128, 128)
    dst2d = jnp.concatenate([edge_index[1], pad_idx]).reshape(EP // 128, 128)
    xp = jnp.concatenate([x, jnp.zeros((NP - NN, DIN), f32)])
    batch3d = jnp.concatenate(
        [batch, jnp.full((NP - NN,), GG, jnp.int32)]).reshape(NB, 1, BR)
    zeros_nh = jnp.zeros((NP, HP), f32)

    bn8 = jnp.broadcast_to(b_node[None, :], (8, HH))
    bg0_8 = jnp.broadcast_to(bg0[None, :], (8, HH))
    bg1_8 = jnp.broadcast_to(bg1[None, :], (8, HH))
    bg2_8 = jnp.broadcast_to(bg2[None, :], (8, HH))
    bfc1_8 = jnp.broadcast_to(bfc1[None, :], (8, HH))
    w2p = jnp.zeros((HH, 128), f32).at[:, :1].set(Wfc2)
    b2p8 = jnp.broadcast_to(
        jnp.concatenate([bfc2, jnp.zeros((127,), f32)])[None, :], (8, 128))

    degp = _sc_deg(dst2d)
    hwp0, dinv64 = _stage_a(xp, degp, W_node, Wg0, bn8)
    p = _sc_agg(src2d, dst2d, hwp0, zeros_nh)
    hwp1 = _stage_b(p, hwp0, dinv64, Wg1, bg0_8)
    p = _sc_agg(src2d, dst2d, hwp1, zeros_nh)
    hwp2 = _stage_b(p, hwp1, dinv64, Wg2, bg1_8)
    p = _sc_agg(src2d, dst2d, hwp2, zeros_nh)
    outp = _stage_c(p, hwp2, dinv64, bg2_8, batch3d, Wfc1, bfc1_8, w2p, b2p8)
    return outp[:GG, :1]
